# Initial kernel scaffold; baseline (speedup 1.0000x reference)
import jax
import jax.numpy as jnp
from jax import lax
from jax.experimental import pallas as pl
from jax.experimental.pallas import tpu as pltpu
from jax.experimental.pallas import tpu_sc as plsc

N_PLAYLISTS = 20000
N_TRACKS = 80000
NN = N_PLAYLISTS + N_TRACKS
D = 32
H = 16
NE = 1600000
N_LAYERS = 3

CHUNK = 128
NCH = NE // CHUNK
CH_PT = NCH // 16
CH_REM = NCH % 16
ROWS_PT = NN // 16
PIECE = 625
N_PIECES = ROWS_PT // PIECE


def _gcn_body(emb_init, src_r, dst_r, w_r, sum_out, scr_a, scr_b,
              idx_v, dst_v, w_v, rows_v, zeros_v, st_acc, st_sum, acc, sem):
    c = lax.axis_index("c")
    s = lax.axis_index("s")
    row_off = c * NN
    base_ch = s * CH_PT + jnp.minimum(s, CH_REM)
    n_ch = CH_PT + jnp.where(s < CH_REM, 1, 0)
    row0 = s * ROWS_PT

    def zfill(j, carry):
        zeros_v[j, :] = jnp.zeros((H,), jnp.float32)
        return carry
    lax.fori_loop(0, PIECE, zfill, 0)

    layer_in = [emb_init, scr_a, scr_b]
    layer_out = [scr_a, scr_b, None]

    for layer in range(N_LAYERS):
        emb_ref = layer_in[layer]
        out_ref = layer_out[layer]

        for p in range(N_PIECES):
            pltpu.sync_copy(zeros_v, acc.at[pl.ds(row0 + p * PIECE, PIECE)])
        plsc.subcore_barrier()

        def chunk_body(i, carry):
            ebase = i * CHUNK
            pltpu.sync_copy(src_r.at[pl.ds(ebase, CHUNK)], idx_v)
            pltpu.sync_copy(dst_r.at[pl.ds(ebase, CHUNK)], dst_v)
            pltpu.sync_copy(w_r.at[pl.ds(ebase, CHUNK)], w_v)
            for g in range(CHUNK // 16):
                sl = pl.ds(g * 16, 16)
                idx_v[sl] = idx_v[sl] + row_off
            pltpu.async_copy(emb_ref.at[idx_v], rows_v, sem).wait()
            for e in range(CHUNK):
                rows_v[e, :] = rows_v[e, :] * w_v[e]
            pltpu.sync_copy(rows_v, acc.at[dst_v], add=True)
            return carry
        lax.fori_loop(base_ch, base_ch + n_ch, chunk_body, 0)
        plsc.subcore_barrier()

        for p in range(N_PIECES):
            r = row0 + p * PIECE
            hr = pl.ds(row_off + r, PIECE)
            if out_ref is not None:
                pltpu.sync_copy(acc.at[pl.ds(r, PIECE)], out_ref.at[hr])
            pltpu.sync_copy(acc.at[pl.ds(r, PIECE)], st_acc)
            if layer == 0:
                pltpu.sync_copy(emb_init.at[hr], st_sum)
            else:
                pltpu.sync_copy(sum_out.at[hr], st_sum)

            def addp(j, carry):
                if layer == N_LAYERS - 1:
                    st_sum[j, :] = (st_sum[j, :] + st_acc[j, :]) * 0.25
                else:
                    st_sum[j, :] = st_sum[j, :] + st_acc[j, :]
                return carry
            lax.fori_loop(0, PIECE, addp, 0)
            pltpu.sync_copy(st_sum, sum_out.at[hr])
        plsc.subcore_barrier()


@jax.jit
def _gcn(emb_init, src, dst, w):
    mesh = plsc.VectorSubcoreMesh(core_axis_name="c", subcore_axis_name="s")
    f = pl.kernel(
        _gcn_body,
        out_type=(
            jax.ShapeDtypeStruct((2 * NN, H), jnp.float32),
            jax.ShapeDtypeStruct((2 * NN, H), jnp.float32),
            jax.ShapeDtypeStruct((2 * NN, H), jnp.float32),
        ),
        mesh=mesh,
        scratch_types=[
            pltpu.VMEM((CHUNK,), jnp.int32),
            pltpu.VMEM((CHUNK,), jnp.int32),
            pltpu.VMEM((CHUNK,), jnp.float32),
            pltpu.VMEM((CHUNK, H), jnp.float32),
            pltpu.VMEM((PIECE, H), jnp.float32),
            pltpu.VMEM((PIECE, H), jnp.float32),
            pltpu.VMEM((PIECE, H), jnp.float32),
            pltpu.VMEM_SHARED((NN, H), jnp.float32),
            pltpu.SemaphoreType.DMA,
        ],
    )
    return f(emb_init, src, dst, w)


def kernel(playlist_w, track_w, edge_weight, edge_index):
    all_emb = jnp.concatenate([playlist_w, track_w], axis=0)
    emb_init = jnp.concatenate([all_emb[:, :H], all_emb[:, H:]], axis=0)
    sum_out, _, _ = _gcn(emb_init, edge_index[0], edge_index[1], edge_weight)
    final = jnp.concatenate([sum_out[:NN], sum_out[NN:]], axis=1)
    return final[:N_PLAYLISTS], final[N_PLAYLISTS:]


# speedup vs baseline: 3.6961x; 3.6961x over previous
import jax
import jax.numpy as jnp
from jax import lax
from jax.experimental import pallas as pl
from jax.experimental.pallas import tpu as pltpu
from jax.experimental.pallas import tpu_sc as plsc

N_PLAYLISTS = 20000
N_TRACKS = 80000
NN = N_PLAYLISTS + N_TRACKS
D = 32
H = 16
NE = 1600000
N_LAYERS = 3

CHUNK = 128
NCH = NE // CHUNK
CH_PT = NCH // 16
CH_REM = NCH % 16
PIECE = 200
PIECES_TOTAL = NN // PIECE
P_PT = PIECES_TOTAL // 16
P_REM = PIECES_TOTAL % 16


def _gcn_body(emb_init, src_r, dst_r, w_r, sum_out, scr_a, scr_b,
              idx_v, dst_v, w_v, rows_v, zeros_v, st_acc, st_sum, acc, sem):
    c = lax.axis_index("c")
    s = lax.axis_index("s")
    row_off = pl.multiple_of(c * NN, 8)
    base_ch = s * CH_PT + jnp.minimum(s, CH_REM)
    n_ch = CH_PT + jnp.where(s < CH_REM, 1, 0)
    pbase = s * P_PT + jnp.minimum(s, P_REM)
    n_p = P_PT + jnp.where(s < P_REM, 1, 0)

    def zfill(j, carry):
        zeros_v[j, :] = jnp.zeros((H,), jnp.float32)
        return carry
    lax.fori_loop(0, PIECE, zfill, 0)

    layer_in = [emb_init, scr_a, scr_b]
    layer_out = [scr_a, scr_b, None]

    for layer in range(N_LAYERS):
        emb_ref = layer_in[layer]
        out_ref = layer_out[layer]

        def zero_body(pidx, carry):
            r = pl.multiple_of(pidx * PIECE, 8)
            pltpu.sync_copy(zeros_v, acc.at[pl.ds(r, PIECE)])
            return carry
        lax.fori_loop(pbase, pbase + n_p, zero_body, 0)
        plsc.subcore_barrier()

        def chunk_body(i, carry):
            ebase = i * CHUNK
            pltpu.sync_copy(src_r.at[pl.ds(ebase, CHUNK)], idx_v)
            pltpu.sync_copy(dst_r.at[pl.ds(ebase, CHUNK)], dst_v)
            pltpu.sync_copy(w_r.at[pl.ds(ebase, CHUNK)], w_v)
            for g in range(CHUNK // 16):
                sl = pl.ds(g * 16, 16)
                idx_v[sl] = idx_v[sl] + row_off
            pltpu.async_copy(emb_ref.at[idx_v], rows_v, sem).wait()
            for g in range(CHUNK // 16):
                wv16 = w_v[pl.ds(g * 16, 16)]
                for e in range(16):
                    r = g * 16 + e
                    rows_v[r, :] = rows_v[r, :] * wv16[e]
            pltpu.sync_copy(rows_v, acc.at[dst_v], add=True)
            return carry
        lax.fori_loop(base_ch, base_ch + n_ch, chunk_body, 0)
        plsc.subcore_barrier()

        def drain_body(pidx, carry):
            r = pl.multiple_of(pidx * PIECE, 8)
            hr = pl.ds(pl.multiple_of(row_off + r, 8), PIECE)
            if out_ref is not None:
                pltpu.sync_copy(acc.at[pl.ds(r, PIECE)], out_ref.at[hr])
            pltpu.sync_copy(acc.at[pl.ds(r, PIECE)], st_acc)
            if layer == 0:
                pltpu.sync_copy(emb_init.at[hr], st_sum)
            else:
                pltpu.sync_copy(sum_out.at[hr], st_sum)

            def addp(j, carry2):
                if layer == N_LAYERS - 1:
                    st_sum[j, :] = (st_sum[j, :] + st_acc[j, :]) * 0.25
                else:
                    st_sum[j, :] = st_sum[j, :] + st_acc[j, :]
                return carry2
            lax.fori_loop(0, PIECE, addp, 0)
            pltpu.sync_copy(st_sum, sum_out.at[hr])
            return carry
        lax.fori_loop(pbase, pbase + n_p, drain_body, 0)
        plsc.subcore_barrier()


@jax.jit
def _gcn(emb_init, src, dst, w):
    mesh = plsc.VectorSubcoreMesh(core_axis_name="c", subcore_axis_name="s")
    f = pl.kernel(
        _gcn_body,
        out_type=(
            jax.ShapeDtypeStruct((2 * NN, H), jnp.float32),
            jax.ShapeDtypeStruct((2 * NN, H), jnp.float32),
            jax.ShapeDtypeStruct((2 * NN, H), jnp.float32),
        ),
        mesh=mesh,
        compiler_params=pltpu.CompilerParams(use_tc_tiling_on_sc=False),
        scratch_types=[
            pltpu.VMEM((CHUNK,), jnp.int32),
            pltpu.VMEM((CHUNK,), jnp.int32),
            pltpu.VMEM((CHUNK,), jnp.float32),
            pltpu.VMEM((CHUNK, H), jnp.float32),
            pltpu.VMEM((PIECE, H), jnp.float32),
            pltpu.VMEM((PIECE, H), jnp.float32),
            pltpu.VMEM((PIECE, H), jnp.float32),
            pltpu.VMEM_SHARED((NN, H), jnp.float32),
            pltpu.SemaphoreType.DMA,
        ],
    )
    return f(emb_init, src, dst, w)


def kernel(playlist_w, track_w, edge_weight, edge_index):
    all_emb = jnp.concatenate([playlist_w, track_w], axis=0)
    emb_init = jnp.concatenate([all_emb[:, :H], all_emb[:, H:]], axis=0)
    sum_out, _, _ = _gcn(emb_init, edge_index[0], edge_index[1], edge_weight)
    final = jnp.concatenate([sum_out[:NN], sum_out[NN:]], axis=1)
    return final[:N_PLAYLISTS], final[N_PLAYLISTS:]


# speedup vs baseline: 8.5360x; 2.3095x over previous
import jax
import jax.numpy as jnp
from jax import lax
from jax.experimental import pallas as pl
from jax.experimental.pallas import tpu as pltpu
from jax.experimental.pallas import tpu_sc as plsc

N_PLAYLISTS = 20000
N_TRACKS = 80000
NN = N_PLAYLISTS + N_TRACKS
D = 32
H = 16
NE = 1600000
N_LAYERS = 3

CHUNK = 512
NCH = NE // CHUNK
CH_PT = NCH // 16
CH_REM = NCH % 16
PIECE = 200
PIECES_TOTAL = NN // PIECE
P_PT = PIECES_TOTAL // 16
P_REM = PIECES_TOTAL % 16


def _gcn_body(emb_init, src_r, dst_r, w_r, sum_out, scr_a, scr_b,
              idx_v, dst_v, w_v, rows_v, zeros_v, st_acc, st_sum, acc, sem):
    c = lax.axis_index("c")
    s = lax.axis_index("s")
    row_off = pl.multiple_of(c * NN, 8)
    base_ch = s * CH_PT + jnp.minimum(s, CH_REM)
    n_ch = CH_PT + jnp.where(s < CH_REM, 1, 0)
    pbase = s * P_PT + jnp.minimum(s, P_REM)
    n_p = P_PT + jnp.where(s < P_REM, 1, 0)

    def zfill(j, carry):
        zeros_v[j, :] = jnp.zeros((H,), jnp.float32)
        return carry
    lax.fori_loop(0, PIECE, zfill, 0)

    layer_in = [emb_init, scr_a, scr_b]
    layer_out = [scr_a, scr_b, None]

    for layer in range(N_LAYERS):
        emb_ref = layer_in[layer]
        out_ref = layer_out[layer]

        def zero_body(pidx, carry):
            r = pl.multiple_of(pidx * PIECE, 8)
            pltpu.sync_copy(zeros_v, acc.at[pl.ds(r, PIECE)])
            return carry
        lax.fori_loop(pbase, pbase + n_p, zero_body, 0)
        plsc.subcore_barrier()

        def chunk_body(i, carry):
            ebase = i * CHUNK
            pltpu.sync_copy(src_r.at[pl.ds(ebase, CHUNK)], idx_v)
            pltpu.sync_copy(dst_r.at[pl.ds(ebase, CHUNK)], dst_v)
            pltpu.sync_copy(w_r.at[pl.ds(ebase, CHUNK)], w_v)
            for g in range(CHUNK // 16):
                sl = pl.ds(g * 16, 16)
                idx_v[sl] = idx_v[sl] + row_off
            pltpu.async_copy(emb_ref.at[idx_v], rows_v, sem).wait()
            for g in range(CHUNK // 16):
                wv16 = w_v[pl.ds(g * 16, 16)]
                for e in range(16):
                    r = g * 16 + e
                    rows_v[r, :] = rows_v[r, :] * wv16[e]
            pltpu.sync_copy(rows_v, acc.at[dst_v], add=True)
            return carry
        lax.fori_loop(base_ch, base_ch + n_ch, chunk_body, 0)
        plsc.subcore_barrier()

        def drain_body(pidx, carry):
            r = pl.multiple_of(pidx * PIECE, 8)
            hr = pl.ds(pl.multiple_of(row_off + r, 8), PIECE)
            if out_ref is not None:
                pltpu.sync_copy(acc.at[pl.ds(r, PIECE)], out_ref.at[hr])
            pltpu.sync_copy(acc.at[pl.ds(r, PIECE)], st_acc)
            if layer == 0:
                pltpu.sync_copy(emb_init.at[hr], st_sum)
            else:
                pltpu.sync_copy(sum_out.at[hr], st_sum)

            def addp(j, carry2):
                if layer == N_LAYERS - 1:
                    st_sum[j, :] = (st_sum[j, :] + st_acc[j, :]) * 0.25
                else:
                    st_sum[j, :] = st_sum[j, :] + st_acc[j, :]
                return carry2
            lax.fori_loop(0, PIECE, addp, 0)
            pltpu.sync_copy(st_sum, sum_out.at[hr])
            return carry
        lax.fori_loop(pbase, pbase + n_p, drain_body, 0)
        plsc.subcore_barrier()


@jax.jit
def _gcn(emb_init, src, dst, w):
    mesh = plsc.VectorSubcoreMesh(core_axis_name="c", subcore_axis_name="s")
    f = pl.kernel(
        _gcn_body,
        out_type=(
            jax.ShapeDtypeStruct((2 * NN, H), jnp.float32),
            jax.ShapeDtypeStruct((2 * NN, H), jnp.float32),
            jax.ShapeDtypeStruct((2 * NN, H), jnp.float32),
        ),
        mesh=mesh,
        compiler_params=pltpu.CompilerParams(use_tc_tiling_on_sc=False),
        scratch_types=[
            pltpu.VMEM((CHUNK,), jnp.int32),
            pltpu.VMEM((CHUNK,), jnp.int32),
            pltpu.VMEM((CHUNK,), jnp.float32),
            pltpu.VMEM((CHUNK, H), jnp.float32),
            pltpu.VMEM((PIECE, H), jnp.float32),
            pltpu.VMEM((PIECE, H), jnp.float32),
            pltpu.VMEM((PIECE, H), jnp.float32),
            pltpu.VMEM_SHARED((NN, H), jnp.float32),
            pltpu.SemaphoreType.DMA,
        ],
    )
    return f(emb_init, src, dst, w)


def kernel(playlist_w, track_w, edge_weight, edge_index):
    all_emb = jnp.concatenate([playlist_w, track_w], axis=0)
    emb_init = jnp.concatenate([all_emb[:, :H], all_emb[:, H:]], axis=0)
    sum_out, _, _ = _gcn(emb_init, edge_index[0], edge_index[1], edge_weight)
    final = jnp.concatenate([sum_out[:NN], sum_out[NN:]], axis=1)
    return final[:N_PLAYLISTS], final[N_PLAYLISTS:]


# speedup vs baseline: 11.1102x; 1.3016x over previous
import jax
import jax.numpy as jnp
from jax import lax
from jax.experimental import pallas as pl
from jax.experimental.pallas import tpu as pltpu
from jax.experimental.pallas import tpu_sc as plsc

N_PLAYLISTS = 20000
N_TRACKS = 80000
NN = N_PLAYLISTS + N_TRACKS
D = 32
H = 16
NE = 1600000
N_LAYERS = 3

CHUNK = 1024
NCH = NE // CHUNK
CH_PT = NCH // 16
CH_REM = NCH % 16
PIECE = 200
PIECES_TOTAL = NN // PIECE
P_PT = PIECES_TOTAL // 16
P_REM = PIECES_TOTAL % 16


def _gcn_body(emb_init, src_r, dst_r, w_r, sum_out, scr_a, scr_b,
              idx_v, dst_v, w_v, rows_v, st_acc, st_sum, acc, sem):
    c = lax.axis_index("c")
    s = lax.axis_index("s")
    row_off = pl.multiple_of(c * NN, 8)
    base_ch = s * CH_PT + jnp.minimum(s, CH_REM)
    n_ch = CH_PT + jnp.where(s < CH_REM, 1, 0)
    pbase = s * P_PT + jnp.minimum(s, P_REM)
    n_p = P_PT + jnp.where(s < P_REM, 1, 0)

    layer_in = [emb_init, scr_a, scr_b]
    layer_out = [scr_a, scr_b, None]

    for layer in range(N_LAYERS):
        emb_ref = layer_in[layer]
        out_ref = layer_out[layer]

        def zfill(j, carry):
            st_acc[j, :] = jnp.zeros((H,), jnp.float32)
            return carry
        lax.fori_loop(0, PIECE, zfill, 0)

        def zero_body(pidx, carry):
            r = pl.multiple_of(pidx * PIECE, 8)
            pltpu.sync_copy(st_acc, acc.at[pl.ds(r, PIECE)])
            return carry
        lax.fori_loop(pbase, pbase + n_p, zero_body, 0)
        plsc.subcore_barrier()

        def chunk_body(i, carry):
            ebase = i * CHUNK
            pltpu.sync_copy(src_r.at[pl.ds(ebase, CHUNK)], idx_v)
            pltpu.sync_copy(dst_r.at[pl.ds(ebase, CHUNK)], dst_v)
            pltpu.sync_copy(w_r.at[pl.ds(ebase, CHUNK)], w_v)
            for g in range(CHUNK // 16):
                sl = pl.ds(g * 16, 16)
                idx_v[sl] = idx_v[sl] + row_off
            pltpu.async_copy(emb_ref.at[idx_v], rows_v, sem).wait()
            for g in range(CHUNK // 16):
                wv16 = w_v[pl.ds(g * 16, 16)]
                for e in range(16):
                    r = g * 16 + e
                    rows_v[r, :] = rows_v[r, :] * wv16[e]
            pltpu.sync_copy(rows_v, acc.at[dst_v], add=True)
            return carry
        lax.fori_loop(base_ch, base_ch + n_ch, chunk_body, 0)
        plsc.subcore_barrier()

        def drain_body(pidx, carry):
            r = pl.multiple_of(pidx * PIECE, 8)
            hr = pl.ds(pl.multiple_of(row_off + r, 8), PIECE)
            if out_ref is not None:
                pltpu.sync_copy(acc.at[pl.ds(r, PIECE)], out_ref.at[hr])
            pltpu.sync_copy(acc.at[pl.ds(r, PIECE)], st_acc)
            if layer == 0:
                pltpu.sync_copy(emb_init.at[hr], st_sum)
            else:
                pltpu.sync_copy(sum_out.at[hr], st_sum)

            def addp(j, carry2):
                if layer == N_LAYERS - 1:
                    st_sum[j, :] = (st_sum[j, :] + st_acc[j, :]) * 0.25
                else:
                    st_sum[j, :] = st_sum[j, :] + st_acc[j, :]
                return carry2
            lax.fori_loop(0, PIECE, addp, 0)
            pltpu.sync_copy(st_sum, sum_out.at[hr])
            return carry
        lax.fori_loop(pbase, pbase + n_p, drain_body, 0)
        plsc.subcore_barrier()


@jax.jit
def _gcn(emb_init, src, dst, w):
    mesh = plsc.VectorSubcoreMesh(core_axis_name="c", subcore_axis_name="s")
    f = pl.kernel(
        _gcn_body,
        out_type=(
            jax.ShapeDtypeStruct((2 * NN, H), jnp.float32),
            jax.ShapeDtypeStruct((2 * NN, H), jnp.float32),
            jax.ShapeDtypeStruct((2 * NN, H), jnp.float32),
        ),
        mesh=mesh,
        compiler_params=pltpu.CompilerParams(use_tc_tiling_on_sc=False),
        scratch_types=[
            pltpu.VMEM((CHUNK,), jnp.int32),
            pltpu.VMEM((CHUNK,), jnp.int32),
            pltpu.VMEM((CHUNK,), jnp.float32),
            pltpu.VMEM((CHUNK, H), jnp.float32),
            pltpu.VMEM((PIECE, H), jnp.float32),
            pltpu.VMEM((PIECE, H), jnp.float32),
            pltpu.VMEM_SHARED((NN, H), jnp.float32),
            pltpu.SemaphoreType.DMA,
        ],
    )
    return f(emb_init, src, dst, w)


def kernel(playlist_w, track_w, edge_weight, edge_index):
    all_emb = jnp.concatenate([playlist_w, track_w], axis=0)
    emb_init = jnp.concatenate([all_emb[:, :H], all_emb[:, H:]], axis=0)
    sum_out, _, _ = _gcn(emb_init, edge_index[0], edge_index[1], edge_weight)
    final = jnp.concatenate([sum_out[:NN], sum_out[NN:]], axis=1)
    return final[:N_PLAYLISTS], final[N_PLAYLISTS:]


# speedup vs baseline: 14.5510x; 1.3097x over previous
import jax
import jax.numpy as jnp
from jax import lax
from jax.experimental import pallas as pl
from jax.experimental.pallas import tpu as pltpu
from jax.experimental.pallas import tpu_sc as plsc

N_PLAYLISTS = 20000
N_TRACKS = 80000
NN = N_PLAYLISTS + N_TRACKS
D = 32
H = 16
NE = 1600000
N_LAYERS = 3

CHUNK = 512
NCH = NE // CHUNK
CH_PT = NCH // 16
CH_REM = NCH % 16
PIECE = 200
PIECES_TOTAL = NN // PIECE
P_PT = PIECES_TOTAL // 16
P_REM = PIECES_TOTAL % 16


def _gcn_body(emb_init, src_r, dst_r, w_r, sum_out, scr_a, scr_b,
              idx0, dst0, w0, rows0, idx1, dst1, w1, rows1,
              st_acc, st_sum, acc, se0, sg0, ss0, se1, sg1, ss1):
    c = lax.axis_index("c")
    s = lax.axis_index("s")
    row_off = pl.multiple_of(c * NN, 8)
    base_ch = s * CH_PT + jnp.minimum(s, CH_REM)
    n_ch = CH_PT + jnp.where(s < CH_REM, 1, 0)
    end_ch = base_ch + n_ch
    pbase = s * P_PT + jnp.minimum(s, P_REM)
    n_p = P_PT + jnp.where(s < P_REM, 1, 0)

    B0 = (idx0, dst0, w0, rows0, se0, sg0, ss0)
    B1 = (idx1, dst1, w1, rows1, se1, sg1, ss1)

    def for_parity(even, fn):
        pl.when(even)(lambda: fn(B0))
        pl.when(jnp.logical_not(even))(lambda: fn(B1))

    def fire_edges(i, b):
        idx, dst, w, rows, se, sg, ss = b
        eb = pl.multiple_of(i * CHUNK, 8)
        pltpu.async_copy(src_r.at[pl.ds(eb, CHUNK)], idx, se)
        pltpu.async_copy(dst_r.at[pl.ds(eb, CHUNK)], dst, se)
        pltpu.async_copy(w_r.at[pl.ds(eb, CHUNK)], w, se)

    def wait_edges(b):
        idx, dst, w, rows, se, sg, ss = b
        pltpu.make_async_copy(src_r.at[pl.ds(0, CHUNK)], idx, se).wait()
        pltpu.make_async_copy(dst_r.at[pl.ds(0, CHUNK)], dst, se).wait()
        pltpu.make_async_copy(w_r.at[pl.ds(0, CHUNK)], w, se).wait()

    def adjust_fire_gather(b, emb_ref):
        idx, dst, w, rows, se, sg, ss = b
        for g in range(CHUNK // 16):
            sl = pl.ds(g * 16, 16)
            idx[sl] = idx[sl] + row_off
        pltpu.async_copy(emb_ref.at[idx], rows, sg)

    def wait_gather(b, emb_ref):
        idx, dst, w, rows, se, sg, ss = b
        pltpu.make_async_copy(emb_ref.at[idx], rows, sg).wait()

    def scale_fire_scatter(b):
        idx, dst, w, rows, se, sg, ss = b
        for g in range(CHUNK // 16):
            wv16 = w[pl.ds(g * 16, 16)]
            for e in range(16):
                r = g * 16 + e
                rows[r, :] = rows[r, :] * wv16[e]
        pltpu.async_copy(rows, acc.at[dst], ss, add=True)

    def wait_scatter(b):
        idx, dst, w, rows, se, sg, ss = b
        pltpu.make_async_copy(rows, acc.at[dst], ss).wait()

    layer_in = [emb_init, scr_a, scr_b]
    layer_out = [scr_a, scr_b, None]

    for layer in range(N_LAYERS):
        emb_ref = layer_in[layer]
        out_ref = layer_out[layer]

        def zfill(j, carry):
            st_acc[j, :] = jnp.zeros((H,), jnp.float32)
            return carry
        lax.fori_loop(0, PIECE, zfill, 0)

        def zero_body(pidx, carry):
            r = pl.multiple_of(pidx * PIECE, 8)
            pltpu.sync_copy(st_acc, acc.at[pl.ds(r, PIECE)])
            return carry
        lax.fori_loop(pbase, pbase + n_p, zero_body, 0)
        plsc.subcore_barrier()

        pe0 = (base_ch % 2) == 0
        for_parity(pe0, lambda b: fire_edges(base_ch, b))
        for_parity(pe0, wait_edges)
        for_parity(pe0, lambda b: adjust_fire_gather(b, emb_ref))

        def chunk_body(i, carry):
            pe = (i % 2) == 0
            po = jnp.logical_not(pe)

            @pl.when(i > base_ch)
            def _():
                for_parity(po, wait_scatter)

            @pl.when(i + 1 < end_ch)
            def _():
                for_parity(po, lambda b: fire_edges(i + 1, b))

            for_parity(pe, lambda b: wait_gather(b, emb_ref))
            for_parity(pe, scale_fire_scatter)

            @pl.when(i + 1 < end_ch)
            def _():
                for_parity(po, wait_edges)
                for_parity(po, lambda b: adjust_fire_gather(b, emb_ref))
            return carry
        lax.fori_loop(base_ch, end_ch, chunk_body, 0)
        p_last = ((end_ch - 1) % 2) == 0
        for_parity(p_last, wait_scatter)
        plsc.subcore_barrier()

        def drain_body(pidx, carry):
            r = pl.multiple_of(pidx * PIECE, 8)
            hr = pl.ds(pl.multiple_of(row_off + r, 8), PIECE)
            if out_ref is not None:
                pltpu.sync_copy(acc.at[pl.ds(r, PIECE)], out_ref.at[hr])
            pltpu.sync_copy(acc.at[pl.ds(r, PIECE)], st_acc)
            if layer == 0:
                pltpu.sync_copy(emb_init.at[hr], st_sum)
            else:
                pltpu.sync_copy(sum_out.at[hr], st_sum)

            def addp(j, carry2):
                if layer == N_LAYERS - 1:
                    st_sum[j, :] = (st_sum[j, :] + st_acc[j, :]) * 0.25
                else:
                    st_sum[j, :] = st_sum[j, :] + st_acc[j, :]
                return carry2
            lax.fori_loop(0, PIECE, addp, 0)
            pltpu.sync_copy(st_sum, sum_out.at[hr])
            return carry
        lax.fori_loop(pbase, pbase + n_p, drain_body, 0)
        plsc.subcore_barrier()


@jax.jit
def _gcn(emb_init, src, dst, w):
    mesh = plsc.VectorSubcoreMesh(core_axis_name="c", subcore_axis_name="s")
    f = pl.kernel(
        _gcn_body,
        out_type=(
            jax.ShapeDtypeStruct((2 * NN, H), jnp.float32),
            jax.ShapeDtypeStruct((2 * NN, H), jnp.float32),
            jax.ShapeDtypeStruct((2 * NN, H), jnp.float32),
        ),
        mesh=mesh,
        compiler_params=pltpu.CompilerParams(use_tc_tiling_on_sc=False),
        scratch_types=(
            [pltpu.VMEM((CHUNK,), jnp.int32),
             pltpu.VMEM((CHUNK,), jnp.int32),
             pltpu.VMEM((CHUNK,), jnp.float32),
             pltpu.VMEM((CHUNK, H), jnp.float32),
             ] * 2
            + [pltpu.VMEM((PIECE, H), jnp.float32),
               pltpu.VMEM((PIECE, H), jnp.float32),
               pltpu.VMEM_SHARED((NN, H), jnp.float32)]
            + [pltpu.SemaphoreType.DMA] * 6
        ),
    )
    return f(emb_init, src, dst, w)


def kernel(playlist_w, track_w, edge_weight, edge_index):
    all_emb = jnp.concatenate([playlist_w, track_w], axis=0)
    emb_init = jnp.concatenate([all_emb[:, :H], all_emb[:, H:]], axis=0)
    sum_out, _, _ = _gcn(emb_init, edge_index[0], edge_index[1], edge_weight)
    final = jnp.concatenate([sum_out[:NN], sum_out[NN:]], axis=1)
    return final[:N_PLAYLISTS], final[N_PLAYLISTS:]


# speedup vs baseline: 14.6268x; 1.0052x over previous
import jax
import jax.numpy as jnp
from jax import lax
from jax.experimental import pallas as pl
from jax.experimental.pallas import tpu as pltpu
from jax.experimental.pallas import tpu_sc as plsc

N_PLAYLISTS = 20000
N_TRACKS = 80000
NN = N_PLAYLISTS + N_TRACKS
D = 32
H = 16
NE = 1600000
N_LAYERS = 3

CHUNK = 512
NCH = NE // CHUNK
CH_PT = NCH // 16
CH_REM = NCH % 16
PIECE = 200
PIECES_TOTAL = NN // PIECE
P_PT = PIECES_TOTAL // 16
P_REM = PIECES_TOTAL % 16


def _gcn_body(emb_init, src_r, dst_r, w_r, sum_out, scr_a, scr_b,
              idx0, dst0, w0, rows0, idx1, dst1, w1, rows1,
              st_acc, st_sum, zeros_v, acc, se0, sg0, ss0, se1, sg1, ss1):
    c = lax.axis_index("c")
    s = lax.axis_index("s")
    row_off = pl.multiple_of(c * NN, 8)
    base_ch = s * CH_PT + jnp.minimum(s, CH_REM)
    n_ch = CH_PT + jnp.where(s < CH_REM, 1, 0)
    end_ch = base_ch + n_ch
    pbase = s * P_PT + jnp.minimum(s, P_REM)
    n_p = P_PT + jnp.where(s < P_REM, 1, 0)

    B0 = (idx0, dst0, w0, rows0, se0, sg0, ss0)
    B1 = (idx1, dst1, w1, rows1, se1, sg1, ss1)

    def for_parity(even, fn):
        pl.when(even)(lambda: fn(B0))
        pl.when(jnp.logical_not(even))(lambda: fn(B1))

    def fire_edges(i, b):
        idx, dst, w, rows, se, sg, ss = b
        eb = pl.multiple_of(i * CHUNK, 8)
        pltpu.async_copy(src_r.at[pl.ds(eb, CHUNK)], idx, se)
        pltpu.async_copy(dst_r.at[pl.ds(eb, CHUNK)], dst, se)
        pltpu.async_copy(w_r.at[pl.ds(eb, CHUNK)], w, se)

    def wait_edges(b):
        idx, dst, w, rows, se, sg, ss = b
        pltpu.make_async_copy(src_r.at[pl.ds(0, CHUNK)], idx, se).wait()
        pltpu.make_async_copy(dst_r.at[pl.ds(0, CHUNK)], dst, se).wait()
        pltpu.make_async_copy(w_r.at[pl.ds(0, CHUNK)], w, se).wait()

    def adjust_fire_gather(b, emb_ref):
        idx, dst, w, rows, se, sg, ss = b
        pltpu.async_copy(emb_ref.at[pl.ds(row_off, NN)].at[idx], rows, sg)

    def wait_gather(b, emb_ref):
        idx, dst, w, rows, se, sg, ss = b
        pltpu.make_async_copy(emb_ref.at[pl.ds(row_off, NN)].at[idx], rows, sg).wait()

    def scale_fire_scatter(b):
        idx, dst, w, rows, se, sg, ss = b
        for g in range(CHUNK // 16):
            wv16 = w[pl.ds(g * 16, 16)]
            for e in range(16):
                r = g * 16 + e
                rows[r, :] = rows[r, :] * wv16[e]
        pltpu.async_copy(rows, acc.at[dst], ss, add=True)

    def wait_scatter(b):
        idx, dst, w, rows, se, sg, ss = b
        pltpu.make_async_copy(rows, acc.at[dst], ss).wait()

    layer_in = [emb_init, scr_a, scr_b]
    layer_out = [scr_a, scr_b, None]

    def zfill(j, carry):
        zeros_v[j, :] = jnp.zeros((H,), jnp.float32)
        return carry
    lax.fori_loop(0, PIECE, zfill, 0)

    def zero_body(pidx, carry):
        r = pl.multiple_of(pidx * PIECE, 8)
        pltpu.sync_copy(zeros_v, acc.at[pl.ds(r, PIECE)])
        return carry
    lax.fori_loop(pbase, pbase + n_p, zero_body, 0)
    plsc.subcore_barrier()

    for layer in range(N_LAYERS):
        emb_ref = layer_in[layer]
        out_ref = layer_out[layer]

        pe0 = (base_ch % 2) == 0
        for_parity(pe0, lambda b: fire_edges(base_ch, b))
        for_parity(pe0, wait_edges)
        for_parity(pe0, lambda b: adjust_fire_gather(b, emb_ref))

        def chunk_body(i, carry):
            pe = (i % 2) == 0
            po = jnp.logical_not(pe)

            @pl.when(i > base_ch)
            def _():
                for_parity(po, wait_scatter)

            @pl.when(i + 1 < end_ch)
            def _():
                for_parity(po, lambda b: fire_edges(i + 1, b))

            for_parity(pe, lambda b: wait_gather(b, emb_ref))
            for_parity(pe, scale_fire_scatter)

            @pl.when(i + 1 < end_ch)
            def _():
                for_parity(po, wait_edges)
                for_parity(po, lambda b: adjust_fire_gather(b, emb_ref))
            return carry
        lax.fori_loop(base_ch, end_ch, chunk_body, 0)
        p_last = ((end_ch - 1) % 2) == 0
        for_parity(p_last, wait_scatter)
        plsc.subcore_barrier()

        def drain_body(pidx, carry):
            r = pl.multiple_of(pidx * PIECE, 8)
            hr = pl.ds(pl.multiple_of(row_off + r, 8), PIECE)
            if out_ref is not None:
                pltpu.sync_copy(acc.at[pl.ds(r, PIECE)], out_ref.at[hr])
            pltpu.sync_copy(acc.at[pl.ds(r, PIECE)], st_acc)
            pltpu.sync_copy(zeros_v, acc.at[pl.ds(r, PIECE)])
            if layer == 0:
                pltpu.sync_copy(emb_init.at[hr], st_sum)
            else:
                pltpu.sync_copy(sum_out.at[hr], st_sum)

            def addp(j, carry2):
                if layer == N_LAYERS - 1:
                    st_sum[j, :] = (st_sum[j, :] + st_acc[j, :]) * 0.25
                else:
                    st_sum[j, :] = st_sum[j, :] + st_acc[j, :]
                return carry2
            lax.fori_loop(0, PIECE, addp, 0)
            pltpu.sync_copy(st_sum, sum_out.at[hr])
            return carry
        lax.fori_loop(pbase, pbase + n_p, drain_body, 0)
        plsc.subcore_barrier()


@jax.jit
def _gcn(emb_init, src, dst, w):
    mesh = plsc.VectorSubcoreMesh(core_axis_name="c", subcore_axis_name="s")
    f = pl.kernel(
        _gcn_body,
        out_type=(
            jax.ShapeDtypeStruct((2 * NN, H), jnp.float32),
            jax.ShapeDtypeStruct((2 * NN, H), jnp.float32),
            jax.ShapeDtypeStruct((2 * NN, H), jnp.float32),
        ),
        mesh=mesh,
        compiler_params=pltpu.CompilerParams(use_tc_tiling_on_sc=False),
        scratch_types=(
            [pltpu.VMEM((CHUNK,), jnp.int32),
             pltpu.VMEM((CHUNK,), jnp.int32),
             pltpu.VMEM((CHUNK,), jnp.float32),
             pltpu.VMEM((CHUNK, H), jnp.float32),
             ] * 2
            + [pltpu.VMEM((PIECE, H), jnp.float32),
               pltpu.VMEM((PIECE, H), jnp.float32),
               pltpu.VMEM((PIECE, H), jnp.float32),
               pltpu.VMEM_SHARED((NN, H), jnp.float32)]
            + [pltpu.SemaphoreType.DMA] * 6
        ),
    )
    return f(emb_init, src, dst, w)


def kernel(playlist_w, track_w, edge_weight, edge_index):
    all_emb = jnp.concatenate([playlist_w, track_w], axis=0)
    emb_init = jnp.concatenate([all_emb[:, :H], all_emb[:, H:]], axis=0)
    sum_out, _, _ = _gcn(emb_init, edge_index[0], edge_index[1], edge_weight)
    final = jnp.concatenate([sum_out[:NN], sum_out[NN:]], axis=1)
    return final[:N_PLAYLISTS], final[N_PLAYLISTS:]
